# baseline (speedup 1.0000x reference)
import functools

import jax
import jax.numpy as jnp
from jax import lax
from jax.experimental import pallas as pl
from jax.experimental.pallas import tpu as pltpu
from jax.experimental.pallas import tpu_sc as plsc

N_NODES = 10000
N_EDGES = 320000
N_FEAT = 128
HIDN = 16
N_CLASSES = 40

NP = 10240
PAD = N_NODES
NC, NS, LANES = 2, 16, 16
NW = NC * NS
CHUNK = 128
NCHUNK_T = 80
NBUF = 8
EPT = NCHUNK_T * CHUNK
EP = NW * EPT
C2 = 48
ROWS_PER_TILE = NP // NS
BLK = 1024


def _mesh():
    return plsc.VectorSubcoreMesh(
        core_axis_name="c", subcore_axis_name="s", num_cores=NC, num_subcores=NS
    )


NRANGE = NS
RNG = NP // NRANGE
CAP = 896


def _sc_route(src_p, dst_p):
    def body(src_hbm, dst_hbm, degp_hbm, srcl_hbm, dstl_hbm,
             src_v, dst_v, deg_v, srcb, dstb, offs):
        c = lax.axis_index("c")
        s = lax.axis_index("s")
        w = s * NC + c
        zero16 = jnp.zeros((LANES,), jnp.float32)
        one16 = jnp.ones((LANES,), jnp.float32)
        padv = jnp.full((LANES,), PAD, jnp.int32)

        pltpu.sync_copy(src_hbm.at[pl.ds(w * EPT, EPT)], src_v)
        pltpu.sync_copy(dst_hbm.at[pl.ds(w * EPT, EPT)], dst_v)

        def zloop(i, _):
            deg_v[pl.ds(i * LANES, LANES)] = zero16
            return None

        lax.fori_loop(0, NP // LANES, zloop, None)

        for r in range(NRANGE):
            basev = jnp.full((LANES,), r * RNG, jnp.int32)

            def pfill(i, _, r=r, basev=basev):
                srcb[r, pl.ds(i * LANES, LANES)] = padv
                dstb[r, pl.ds(i * LANES, LANES)] = basev
                return None

            lax.fori_loop(0, CAP // LANES, pfill, None)
            offs[r] = 0

        def eloop(i, _):
            dstv = dst_v[pl.ds(i * LANES, LANES)]
            srcv = src_v[pl.ds(i * LANES, LANES)]
            plsc.addupdate_scatter(deg_v, [dstv], one16)
            rv = lax.shift_right_logical(dstv, 7)
            rv = lax.shift_right_logical(rv * 13108, 16)
            for r in range(NRANGE):
                m = rv == r
                off = jnp.minimum(offs[r], CAP - LANES)
                plsc.store_compressed(srcb.at[r, pl.ds(off, LANES)], srcv, mask=m)
                plsc.store_compressed(dstb.at[r, pl.ds(off, LANES)], dstv, mask=m)
                offs[r] = off + jnp.sum(m.astype(jnp.int32))
            return None

        lax.fori_loop(0, EPT // LANES, eloop, None)

        pltpu.sync_copy(deg_v, degp_hbm.at[w])
        pltpu.sync_copy(srcb, srcl_hbm.at[w])
        pltpu.sync_copy(dstb, dstl_hbm.at[w])

    fn = pl.kernel(
        body,
        out_type=(
            jax.ShapeDtypeStruct((NW, NP), jnp.float32),
            jax.ShapeDtypeStruct((NW, NRANGE, CAP), jnp.int32),
            jax.ShapeDtypeStruct((NW, NRANGE, CAP), jnp.int32),
        ),
        mesh=_mesh(),
        scratch_types=[
            pltpu.VMEM((EPT,), jnp.int32),
            pltpu.VMEM((EPT,), jnp.int32),
            pltpu.VMEM((NP,), jnp.float32),
            pltpu.VMEM((NRANGE, CAP), jnp.int32),
            pltpu.VMEM((NRANGE, CAP), jnp.int32),
            pltpu.SMEM((NRANGE,), jnp.int32),
        ],
        compiler_params=pltpu.CompilerParams(
            needs_layout_passes=False, use_tc_tiling_on_sc=False
        ),
    )
    return fn(src_p, dst_p)


def _sc_gather_acc(g, srcl, dstl, d):
    nch = CAP // CHUNK
    nk = d // LANES

    def body(g_hbm, srcl_hbm, dstl_hbm, out_hbm,
             src_cells, dst_cells, rows0, rows1, acc, semc, sem0, sem1):
        c = lax.axis_index("c")
        s = lax.axis_index("s")
        zero16 = jnp.zeros((LANES,), jnp.float32)
        base = s * RNG

        for p16 in range(NS):
            p = p16 * NC + c
            pltpu.async_copy(srcl_hbm.at[p, s], src_cells.at[p16], semc)
            pltpu.async_copy(dstl_hbm.at[p, s], dst_cells.at[p16], semc)

        def zacc(i, _):
            for k in range(nk):
                acc[i, pl.ds(k * LANES, LANES)] = zero16
            return None

        lax.fori_loop(0, RNG, zacc, None)

        for p16 in range(NS):
            p = p16 * NC + c
            pltpu.make_async_copy(srcl_hbm.at[p, s], src_cells.at[p16], semc).wait()
            pltpu.make_async_copy(dstl_hbm.at[p, s], dst_cells.at[p16], semc).wait()

        rows = (rows0, rows1)
        sems = (sem0, sem1)

        def sentinel(p16, j):
            head = src_cells[p16, pl.ds(j * CHUNK, LANES)]
            return head[0] != PAD

        def accum(rbuf, p16, j):
            def ebody(ii, _):
                dv = dst_cells[p16, pl.ds(j * CHUNK + ii * LANES, LANES)] - base
                for l in range(LANES):
                    dloc = dv[l]
                    e = ii * LANES + l
                    for k in range(nk):
                        sl = pl.ds(k * LANES, LANES)
                        acc[dloc, sl] = acc[dloc, sl] + rbuf[e, sl]
                return None

            lax.fori_loop(0, CHUNK // LANES, ebody, None)

        def cell(p16, _):
            @pl.when(sentinel(p16, 0))
            def _():
                pltpu.async_copy(
                    g_hbm.at[src_cells.at[p16, pl.ds(0, CHUNK)]], rows[0], sems[0]
                )

            for j in range(nch):
                b = j % 2
                live = sentinel(p16, j)

                @pl.when(live)
                def _(b=b, j=j):
                    pltpu.make_async_copy(
                        g_hbm.at[src_cells.at[p16, pl.ds(j * CHUNK, CHUNK)]],
                        rows[b], sems[b],
                    ).wait()
                    if j + 1 < nch:
                        @pl.when(sentinel(p16, j + 1))
                        def _():
                            pltpu.async_copy(
                                g_hbm.at[src_cells.at[p16, pl.ds((j + 1) * CHUNK, CHUNK)]],
                                rows[1 - b], sems[1 - b],
                            )

                    accum(rows[b], p16, j)

            return None

        lax.fori_loop(0, NS, cell, None)

        pltpu.sync_copy(acc, out_hbm.at[c, pl.ds(base, RNG)])

    fn = pl.kernel(
        body,
        out_type=jax.ShapeDtypeStruct((NC, NP, d), jnp.float32),
        mesh=_mesh(),
        scratch_types=[
            pltpu.VMEM((NS, CAP), jnp.int32),
            pltpu.VMEM((NS, CAP), jnp.int32),
            pltpu.VMEM((CHUNK, d), jnp.float32),
            pltpu.VMEM((CHUNK, d), jnp.float32),
            pltpu.VMEM((RNG, d), jnp.float32),
            pltpu.SemaphoreType.DMA,
            pltpu.SemaphoreType.DMA,
            pltpu.SemaphoreType.DMA,
        ],
        compiler_params=pltpu.CompilerParams(
            needs_layout_passes=False, use_tc_tiling_on_sc=False
        ),
    )
    return fn(g, srcl, dstl)


def _tc_pre_body(degp_ref, data_ref, w1_ref, h1_ref, g1_ref, dis_ref):
    deg = jnp.sum(degp_ref[...], axis=0) + 1.0
    dis = lax.rsqrt(deg)[:, None]
    h = jnp.dot(data_ref[...], w1_ref[...], preferred_element_type=jnp.float32)
    h1_ref[...] = h
    g1_ref[...] = h * dis
    dis_ref[...] = dis


def _tc_pre(degp, data_p, W1):
    grid = NP // BLK
    return pl.pallas_call(
        _tc_pre_body,
        grid=(grid,),
        in_specs=[
            pl.BlockSpec((NW, BLK), lambda i: (0, i)),
            pl.BlockSpec((BLK, N_FEAT), lambda i: (i, 0)),
            pl.BlockSpec((N_FEAT, HIDN), lambda i: (0, 0)),
        ],
        out_specs=[
            pl.BlockSpec((BLK, HIDN), lambda i: (i, 0)),
            pl.BlockSpec((BLK, HIDN), lambda i: (i, 0)),
            pl.BlockSpec((BLK, 1), lambda i: (i, 0)),
        ],
        out_shape=[
            jax.ShapeDtypeStruct((NP, HIDN), jnp.float32),
            jax.ShapeDtypeStruct((NP, HIDN), jnp.float32),
            jax.ShapeDtypeStruct((NP, 1), jnp.float32),
        ],
    )(degp, data_p, W1)


def _tc_mid_body(p1_ref, h1_ref, dis_ref, w2_ref, b1_ref, g2_ref, h2_ref):
    i = pl.program_id(0)
    dis = dis_ref[...]
    acc = p1_ref[0] + p1_ref[1]
    x1 = dis * acc + (dis * dis) * h1_ref[...] + b1_ref[...]
    x1 = jnp.maximum(x1, 0.0)
    rows = i * BLK + lax.broadcasted_iota(jnp.int32, (BLK, 1), 0)
    x1 = jnp.where(rows < N_NODES, x1, 0.0)
    h2 = jnp.dot(x1, w2_ref[...], preferred_element_type=jnp.float32)
    h2_ref[...] = h2
    g2_ref[...] = h2 * dis


def _tc_mid(p1, h1, dis, w2p, b1r):
    grid = NP // BLK
    return pl.pallas_call(
        _tc_mid_body,
        grid=(grid,),
        in_specs=[
            pl.BlockSpec((NC, BLK, HIDN), lambda i: (0, i, 0)),
            pl.BlockSpec((BLK, HIDN), lambda i: (i, 0)),
            pl.BlockSpec((BLK, 1), lambda i: (i, 0)),
            pl.BlockSpec((HIDN, C2), lambda i: (0, 0)),
            pl.BlockSpec((1, HIDN), lambda i: (0, 0)),
        ],
        out_specs=[
            pl.BlockSpec((BLK, C2), lambda i: (i, 0)),
            pl.BlockSpec((BLK, C2), lambda i: (i, 0)),
        ],
        out_shape=[
            jax.ShapeDtypeStruct((NP, C2), jnp.float32),
            jax.ShapeDtypeStruct((NP, C2), jnp.float32),
        ],
    )(p1, h1, dis, w2p, b1r)


def _tc_post_body(p2_ref, h2_ref, dis_ref, b2_ref, out_ref):
    dis = dis_ref[...]
    acc = p2_ref[0] + p2_ref[1]
    out_ref[...] = dis * acc + (dis * dis) * h2_ref[...] + b2_ref[...]


def _tc_post(p2, h2, dis, b2r):
    grid = NP // BLK
    return pl.pallas_call(
        _tc_post_body,
        grid=(grid,),
        in_specs=[
            pl.BlockSpec((NC, BLK, C2), lambda i: (0, i, 0)),
            pl.BlockSpec((BLK, C2), lambda i: (i, 0)),
            pl.BlockSpec((BLK, 1), lambda i: (i, 0)),
            pl.BlockSpec((1, C2), lambda i: (0, 0)),
        ],
        out_specs=pl.BlockSpec((BLK, C2), lambda i: (i, 0)),
        out_shape=jax.ShapeDtypeStruct((NP, C2), jnp.float32),
    )(p2, h2, dis, b2r)


def kernel(data, adj, W1, b1, W2, b2):
    src = adj[0].astype(jnp.int32)
    dst = adj[1].astype(jnp.int32)
    pad = jnp.full((EP - N_EDGES,), PAD, jnp.int32)
    src_p = jnp.concatenate([src, pad])
    dst_p = jnp.concatenate([dst, pad])
    data_p = jnp.zeros((NP, N_FEAT), jnp.float32).at[:N_NODES].set(data)
    w2p = jnp.zeros((HIDN, C2), jnp.float32).at[:, :N_CLASSES].set(W2)
    b1r = b1.reshape(1, HIDN)
    b2r = jnp.zeros((1, C2), jnp.float32).at[0, :N_CLASSES].set(b2)

    degp, srcl, dstl = _sc_route(src_p, dst_p)
    h1, g1, dis = _tc_pre(degp, data_p, W1)
    p1 = _sc_gather_acc(g1, srcl, dstl, HIDN)
    g2, h2 = _tc_mid(p1, h1, dis, w2p, b1r)
    p2 = _sc_gather_acc(g2, srcl, dstl, C2)
    outp = _tc_post(p2, h2, dis, b2r)
    return outp[:N_NODES, :N_CLASSES]


# speedup vs baseline: 3.0706x; 3.0706x over previous
import functools

import jax
import jax.numpy as jnp
from jax import lax
from jax.experimental import pallas as pl
from jax.experimental.pallas import tpu as pltpu
from jax.experimental.pallas import tpu_sc as plsc

N_NODES = 10000
N_EDGES = 320000
N_FEAT = 128
HIDN = 16
N_CLASSES = 40

NP = 10240
PAD = N_NODES
NC, NS, LANES = 2, 16, 16
NW = NC * NS
CHUNK = 128
NCHUNK_T = 80
NBUF = 8
EPT = NCHUNK_T * CHUNK
EP = NW * EPT
C2 = 40
ROWS_PER_TILE = NP // NS
BLK = 1024


def _mesh():
    return plsc.VectorSubcoreMesh(
        core_axis_name="c", subcore_axis_name="s", num_cores=NC, num_subcores=NS
    )


def _sc_deg(adjp):
    def body(adj_hbm, out_hbm, dst_v, deg_v):
        c = lax.axis_index("c")
        s = lax.axis_index("s")
        w = s * NC + c
        zero16 = jnp.zeros((LANES,), jnp.float32)
        one16 = jnp.ones((LANES,), jnp.float32)

        def zloop(i, _):
            deg_v[pl.ds(i * LANES, LANES)] = zero16
            return None

        lax.fori_loop(0, NP // LANES, zloop, None)
        pltpu.sync_copy(adj_hbm.at[1, pl.ds(w * NCHUNK_T, NCHUNK_T)], dst_v)

        def eloop(r, _):
            for jj in range(CHUNK // LANES):
                idx = dst_v[r, pl.ds(jj * LANES, LANES)]
                plsc.addupdate_scatter(deg_v, [idx], one16)
            return None

        lax.fori_loop(0, NCHUNK_T, eloop, None)
        pltpu.sync_copy(deg_v, out_hbm.at[w])

    fn = pl.kernel(
        body,
        out_type=jax.ShapeDtypeStruct((NW, NP), jnp.float32),
        mesh=_mesh(),
        scratch_types=[
            pltpu.VMEM((NCHUNK_T, CHUNK), jnp.int32),
            pltpu.VMEM((NP,), jnp.float32),
        ],
        compiler_params=pltpu.CompilerParams(needs_layout_passes=False),
    )
    return fn(adjp)


def _sc_scatter(g, adjp, zeros_blk, d):
    def body(g_hbm, adj_hbm, zeros_hbm, out_hbm, src_v, dst_v, rows, acc, *sems):
        c = lax.axis_index("c")
        s = lax.axis_index("s")
        w = s * NC + c

        def zacc(k, _):
            pltpu.sync_copy(zeros_hbm, acc.at[pl.ds(s * ROWS_PER_TILE + k * CHUNK, CHUNK)])
            return None

        lax.fori_loop(0, ROWS_PER_TILE // CHUNK, zacc, None)

        pltpu.sync_copy(adj_hbm.at[0, pl.ds(w * NCHUNK_T, NCHUNK_T)], src_v)
        pltpu.sync_copy(adj_hbm.at[1, pl.ds(w * NCHUNK_T, NCHUNK_T)], dst_v)
        plsc.subcore_barrier()

        for b in range(NBUF):
            pltpu.async_copy(g_hbm.at[src_v.at[b]], rows.at[b], sems[b])

        def group(k, _):
            for b in range(NBUF):
                ch = k * NBUF + b
                pltpu.make_async_copy(g_hbm.at[src_v.at[ch]], rows.at[b], sems[b]).wait()
                pltpu.sync_copy(rows.at[b], acc.at[dst_v.at[ch]], add=True)

                @pl.when(ch + NBUF < NCHUNK_T)
                def _():
                    pltpu.async_copy(g_hbm.at[src_v.at[ch + NBUF]], rows.at[b], sems[b])

            return None

        lax.fori_loop(0, NCHUNK_T // NBUF, group, None)
        plsc.subcore_barrier()

        def flush(k, _):
            a = s * ROWS_PER_TILE + k * CHUNK
            pltpu.sync_copy(acc.at[pl.ds(a, CHUNK)], out_hbm.at[c, pl.ds(a, CHUNK)])
            return None

        lax.fori_loop(0, ROWS_PER_TILE // CHUNK, flush, None)

    fn = pl.kernel(
        body,
        out_type=jax.ShapeDtypeStruct((NC, NP, d), jnp.float32),
        mesh=_mesh(),
        scratch_types=[
            pltpu.VMEM((NCHUNK_T, CHUNK), jnp.int32),
            pltpu.VMEM((NCHUNK_T, CHUNK), jnp.int32),
            pltpu.VMEM((NBUF, CHUNK, d), jnp.float32),
            pltpu.VMEM_SHARED((NP, d), jnp.float32),
        ] + [pltpu.SemaphoreType.DMA] * NBUF,
        compiler_params=pltpu.CompilerParams(
            needs_layout_passes=False, use_tc_tiling_on_sc=False
        ),
    )
    return fn(g, adjp, zeros_blk)


def _tc_pre_body(degp_ref, data_ref, w1_ref, h1_ref, g1_ref, dis_ref):
    i = pl.program_id(0)
    deg = jnp.sum(degp_ref[...], axis=0) + 1.0
    dis = lax.rsqrt(deg)[:, None]
    h = jnp.dot(data_ref[...], w1_ref[...], preferred_element_type=jnp.float32)
    rows = i * BLK + lax.broadcasted_iota(jnp.int32, (BLK, 1), 0)
    h1_ref[...] = h
    g1_ref[...] = jnp.where(rows < N_NODES, h * dis, 0.0)
    dis_ref[...] = dis


def _tc_pre(degp, data_p, W1):
    grid = NP // BLK
    return pl.pallas_call(
        _tc_pre_body,
        grid=(grid,),
        in_specs=[
            pl.BlockSpec((NW, BLK), lambda i: (0, i)),
            pl.BlockSpec((BLK, N_FEAT), lambda i: (i, 0)),
            pl.BlockSpec((N_FEAT, HIDN), lambda i: (0, 0)),
        ],
        out_specs=[
            pl.BlockSpec((BLK, HIDN), lambda i: (i, 0)),
            pl.BlockSpec((BLK, HIDN), lambda i: (i, 0)),
            pl.BlockSpec((BLK, 1), lambda i: (i, 0)),
        ],
        out_shape=[
            jax.ShapeDtypeStruct((NP, HIDN), jnp.float32),
            jax.ShapeDtypeStruct((NP, HIDN), jnp.float32),
            jax.ShapeDtypeStruct((NP, 1), jnp.float32),
        ],
    )(degp, data_p, W1)


def _tc_mid_body(p1_ref, h1_ref, dis_ref, w2_ref, b1_ref, g2_ref, h2_ref):
    i = pl.program_id(0)
    dis = dis_ref[...]
    acc = p1_ref[0] + p1_ref[1]
    x1 = dis * acc + (dis * dis) * h1_ref[...] + b1_ref[...]
    x1 = jnp.maximum(x1, 0.0)
    rows = i * BLK + lax.broadcasted_iota(jnp.int32, (BLK, 1), 0)
    x1 = jnp.where(rows < N_NODES, x1, 0.0)
    h2 = jnp.dot(x1, w2_ref[...], preferred_element_type=jnp.float32)
    h2_ref[...] = h2
    g2_ref[...] = h2 * dis


def _tc_mid(p1, h1, dis, w2p, b1r):
    grid = NP // BLK
    return pl.pallas_call(
        _tc_mid_body,
        grid=(grid,),
        in_specs=[
            pl.BlockSpec((NC, BLK, HIDN), lambda i: (0, i, 0)),
            pl.BlockSpec((BLK, HIDN), lambda i: (i, 0)),
            pl.BlockSpec((BLK, 1), lambda i: (i, 0)),
            pl.BlockSpec((HIDN, C2), lambda i: (0, 0)),
            pl.BlockSpec((1, HIDN), lambda i: (0, 0)),
        ],
        out_specs=[
            pl.BlockSpec((BLK, C2), lambda i: (i, 0)),
            pl.BlockSpec((BLK, C2), lambda i: (i, 0)),
        ],
        out_shape=[
            jax.ShapeDtypeStruct((NP, C2), jnp.float32),
            jax.ShapeDtypeStruct((NP, C2), jnp.float32),
        ],
    )(p1, h1, dis, w2p, b1r)


OBLK = 1000


def _tc_post_body(p2_ref, h2_ref, dis_ref, b2_ref, out_ref):
    dis = dis_ref[...]
    acc = p2_ref[0] + p2_ref[1]
    out_ref[...] = dis * acc + (dis * dis) * h2_ref[...] + b2_ref[...]


def _tc_post(p2, h2, dis, b2r):
    grid = N_NODES // OBLK
    return pl.pallas_call(
        _tc_post_body,
        grid=(grid,),
        in_specs=[
            pl.BlockSpec((NC, OBLK, C2), lambda i: (0, i, 0)),
            pl.BlockSpec((OBLK, C2), lambda i: (i, 0)),
            pl.BlockSpec((OBLK, 1), lambda i: (i, 0)),
            pl.BlockSpec((1, C2), lambda i: (0, 0)),
        ],
        out_specs=pl.BlockSpec((OBLK, C2), lambda i: (i, 0)),
        out_shape=jax.ShapeDtypeStruct((N_NODES, C2), jnp.float32),
    )(p2, h2, dis, b2r)


def kernel(data, adj, W1, b1, W2, b2):
    adjp = jnp.concatenate(
        [adj.astype(jnp.int32), jnp.full((2, EP - N_EDGES), PAD, jnp.int32)], axis=1
    ).reshape(2, EP // CHUNK, CHUNK)
    b1r = b1.reshape(1, HIDN)
    b2r = b2.reshape(1, C2)
    z1 = jnp.zeros((CHUNK, HIDN), jnp.float32)
    z2 = jnp.zeros((CHUNK, C2), jnp.float32)

    degp = _sc_deg(adjp)
    h1, g1, dis = _tc_pre(degp, data, W1)
    p1 = _sc_scatter(g1, adjp, z1, HIDN)
    g2, h2 = _tc_mid(p1, h1, dis, W2, b1r)
    p2 = _sc_scatter(g2, adjp, z2, C2)
    return _tc_post(p2, h2, dis, b2r)


# speedup vs baseline: 3.1059x; 1.0115x over previous
import functools

import jax
import jax.numpy as jnp
from jax import lax
from jax.experimental import pallas as pl
from jax.experimental.pallas import tpu as pltpu
from jax.experimental.pallas import tpu_sc as plsc

N_NODES = 10000
N_EDGES = 320000
N_FEAT = 128
HIDN = 16
N_CLASSES = 40

NP = 10240
PAD = N_NODES
NC, NS, LANES = 2, 16, 16
NW = NC * NS
CHUNK = 128
NCHUNK_T = 80
NBUF = 8
EPT = NCHUNK_T * CHUNK
EP = NW * EPT
C2 = 40
ROWS_PER_TILE = NP // NS
BLK = 2048


def _mesh():
    return plsc.VectorSubcoreMesh(
        core_axis_name="c", subcore_axis_name="s", num_cores=NC, num_subcores=NS
    )


def _sc_deg(adjp):
    def body(adj_hbm, out_hbm, dst_v, deg_v):
        c = lax.axis_index("c")
        s = lax.axis_index("s")
        w = s * NC + c
        zero16 = jnp.zeros((LANES,), jnp.float32)
        one16 = jnp.ones((LANES,), jnp.float32)

        def zloop(i, _):
            deg_v[pl.ds(i * LANES, LANES)] = zero16
            return None

        lax.fori_loop(0, NP // LANES, zloop, None)
        pltpu.sync_copy(adj_hbm.at[1, pl.ds(w * NCHUNK_T, NCHUNK_T)], dst_v)

        def eloop(r, _):
            for jj in range(CHUNK // LANES):
                idx = dst_v[r, pl.ds(jj * LANES, LANES)]
                plsc.addupdate_scatter(deg_v, [idx], one16)
            return None

        lax.fori_loop(0, NCHUNK_T, eloop, None)
        pltpu.sync_copy(deg_v, out_hbm.at[w])

    fn = pl.kernel(
        body,
        out_type=jax.ShapeDtypeStruct((NW, NP), jnp.float32),
        mesh=_mesh(),
        scratch_types=[
            pltpu.VMEM((NCHUNK_T, CHUNK), jnp.int32),
            pltpu.VMEM((NP,), jnp.float32),
        ],
        compiler_params=pltpu.CompilerParams(needs_layout_passes=False),
    )
    return fn(adjp)


def _sc_scatter(g, adjp, zeros_blk, d):
    def body(g_hbm, adj_hbm, zeros_hbm, out_hbm, src_v, dst_v, rows, acc, *sems):
        c = lax.axis_index("c")
        s = lax.axis_index("s")
        w = s * NC + c

        def zacc(k, _):
            pltpu.sync_copy(zeros_hbm, acc.at[pl.ds(s * ROWS_PER_TILE + k * CHUNK, CHUNK)])
            return None

        lax.fori_loop(0, ROWS_PER_TILE // CHUNK, zacc, None)

        pltpu.sync_copy(adj_hbm.at[0, pl.ds(w * NCHUNK_T, NCHUNK_T)], src_v)
        pltpu.sync_copy(adj_hbm.at[1, pl.ds(w * NCHUNK_T, NCHUNK_T)], dst_v)
        plsc.subcore_barrier()

        for b in range(NBUF):
            pltpu.async_copy(g_hbm.at[src_v.at[b]], rows.at[b], sems[b])

        def group(k, _):
            for b in range(NBUF):
                ch = k * NBUF + b
                pltpu.make_async_copy(g_hbm.at[src_v.at[ch]], rows.at[b], sems[b]).wait()
                pltpu.sync_copy(rows.at[b], acc.at[dst_v.at[ch]], add=True)

                @pl.when(ch + NBUF < NCHUNK_T)
                def _():
                    pltpu.async_copy(g_hbm.at[src_v.at[ch + NBUF]], rows.at[b], sems[b])

            return None

        lax.fori_loop(0, NCHUNK_T // NBUF, group, None)
        plsc.subcore_barrier()

        def flush(k, _):
            a = s * ROWS_PER_TILE + k * CHUNK
            pltpu.sync_copy(acc.at[pl.ds(a, CHUNK)], out_hbm.at[c, pl.ds(a, CHUNK)])
            return None

        lax.fori_loop(0, ROWS_PER_TILE // CHUNK, flush, None)

    fn = pl.kernel(
        body,
        out_type=jax.ShapeDtypeStruct((NC, NP, d), jnp.float32),
        mesh=_mesh(),
        scratch_types=[
            pltpu.VMEM((NCHUNK_T, CHUNK), jnp.int32),
            pltpu.VMEM((NCHUNK_T, CHUNK), jnp.int32),
            pltpu.VMEM((NBUF, CHUNK, d), jnp.float32),
            pltpu.VMEM_SHARED((NP, d), jnp.float32),
        ] + [pltpu.SemaphoreType.DMA] * NBUF,
        compiler_params=pltpu.CompilerParams(
            needs_layout_passes=False, use_tc_tiling_on_sc=False
        ),
    )
    return fn(g, adjp, zeros_blk)


def _tc_pre_body(degp_ref, data_ref, w1_ref, h1_ref, g1_ref, dis_ref):
    i = pl.program_id(0)
    deg = jnp.sum(degp_ref[...], axis=0) + 1.0
    dis = lax.rsqrt(deg)[:, None]
    h = jnp.dot(data_ref[...], w1_ref[...], preferred_element_type=jnp.float32)
    rows = i * BLK + lax.broadcasted_iota(jnp.int32, (BLK, 1), 0)
    h1_ref[...] = h
    g1_ref[...] = jnp.where(rows < N_NODES, h * dis, 0.0)
    dis_ref[...] = dis


def _tc_pre(degp, data_p, W1):
    grid = NP // BLK
    return pl.pallas_call(
        _tc_pre_body,
        grid=(grid,),
        in_specs=[
            pl.BlockSpec((NW, BLK), lambda i: (0, i)),
            pl.BlockSpec((BLK, N_FEAT), lambda i: (i, 0)),
            pl.BlockSpec((N_FEAT, HIDN), lambda i: (0, 0)),
        ],
        out_specs=[
            pl.BlockSpec((BLK, HIDN), lambda i: (i, 0)),
            pl.BlockSpec((BLK, HIDN), lambda i: (i, 0)),
            pl.BlockSpec((BLK, 1), lambda i: (i, 0)),
        ],
        out_shape=[
            jax.ShapeDtypeStruct((NP, HIDN), jnp.float32),
            jax.ShapeDtypeStruct((NP, HIDN), jnp.float32),
            jax.ShapeDtypeStruct((NP, 1), jnp.float32),
        ],
    )(degp, data_p, W1)


def _tc_mid_body(p1_ref, h1_ref, dis_ref, w2_ref, b1_ref, g2_ref, h2_ref):
    i = pl.program_id(0)
    dis = dis_ref[...]
    acc = p1_ref[0] + p1_ref[1]
    x1 = dis * acc + (dis * dis) * h1_ref[...] + b1_ref[...]
    x1 = jnp.maximum(x1, 0.0)
    rows = i * BLK + lax.broadcasted_iota(jnp.int32, (BLK, 1), 0)
    x1 = jnp.where(rows < N_NODES, x1, 0.0)
    h2 = jnp.dot(x1, w2_ref[...], preferred_element_type=jnp.float32)
    h2_ref[...] = h2
    g2_ref[...] = h2 * dis


def _tc_mid(p1, h1, dis, w2p, b1r):
    grid = NP // BLK
    return pl.pallas_call(
        _tc_mid_body,
        grid=(grid,),
        in_specs=[
            pl.BlockSpec((NC, BLK, HIDN), lambda i: (0, i, 0)),
            pl.BlockSpec((BLK, HIDN), lambda i: (i, 0)),
            pl.BlockSpec((BLK, 1), lambda i: (i, 0)),
            pl.BlockSpec((HIDN, C2), lambda i: (0, 0)),
            pl.BlockSpec((1, HIDN), lambda i: (0, 0)),
        ],
        out_specs=[
            pl.BlockSpec((BLK, C2), lambda i: (i, 0)),
            pl.BlockSpec((BLK, C2), lambda i: (i, 0)),
        ],
        out_shape=[
            jax.ShapeDtypeStruct((NP, C2), jnp.float32),
            jax.ShapeDtypeStruct((NP, C2), jnp.float32),
        ],
    )(p1, h1, dis, w2p, b1r)


OBLK = 1000


def _tc_post_body(p2_ref, h2_ref, dis_ref, b2_ref, out_ref):
    dis = dis_ref[...]
    acc = p2_ref[0] + p2_ref[1]
    out_ref[...] = dis * acc + (dis * dis) * h2_ref[...] + b2_ref[...]


def _tc_post(p2, h2, dis, b2r):
    grid = N_NODES // OBLK
    return pl.pallas_call(
        _tc_post_body,
        grid=(grid,),
        in_specs=[
            pl.BlockSpec((NC, OBLK, C2), lambda i: (0, i, 0)),
            pl.BlockSpec((OBLK, C2), lambda i: (i, 0)),
            pl.BlockSpec((OBLK, 1), lambda i: (i, 0)),
            pl.BlockSpec((1, C2), lambda i: (0, 0)),
        ],
        out_specs=pl.BlockSpec((OBLK, C2), lambda i: (i, 0)),
        out_shape=jax.ShapeDtypeStruct((N_NODES, C2), jnp.float32),
    )(p2, h2, dis, b2r)


def kernel(data, adj, W1, b1, W2, b2):
    adjp = jnp.concatenate(
        [adj.astype(jnp.int32), jnp.full((2, EP - N_EDGES), PAD, jnp.int32)], axis=1
    ).reshape(2, EP // CHUNK, CHUNK)
    b1r = b1.reshape(1, HIDN)
    b2r = b2.reshape(1, C2)
    z1 = jnp.zeros((CHUNK, HIDN), jnp.float32)
    z2 = jnp.zeros((CHUNK, C2), jnp.float32)

    degp = _sc_deg(adjp)
    h1, g1, dis = _tc_pre(degp, data, W1)
    p1 = _sc_scatter(g1, adjp, z1, HIDN)
    g2, h2 = _tc_mid(p1, h1, dis, W2, b1r)
    p2 = _sc_scatter(g2, adjp, z2, C2)
    return _tc_post(p2, h2, dis, b2r)
